# baseline (speedup 1.0000x reference)
import functools

import jax
import jax.numpy as jnp
from jax import lax
from jax.experimental import pallas as pl
from jax.experimental.pallas import tpu as pltpu
from jax.experimental.pallas import tpu_sc as plsc

LAT_R = 256
LAT_C = 256
HALF = LAT_C // 2


@functools.lru_cache(maxsize=None)
def _make_sc_split(num_rows: int):
    info = plsc.get_sparse_core_info()
    NC, NS, L = info.num_cores, info.num_subcores, info.num_lanes
    NW = NC * NS
    rows_per_w = num_rows // NW
    R = 64
    n_chunks = rows_per_w // R
    n_vregs = HALF // L

    mesh = plsc.VectorSubcoreMesh(core_axis_name="c", subcore_axis_name="s")

    @functools.partial(
        pl.kernel,
        mesh=mesh,
        out_type=(
            jax.ShapeDtypeStruct((num_rows * HALF,), jnp.float32),
            jax.ShapeDtypeStruct((num_rows * HALF,), jnp.float32),
        ),
        scratch_types=[
            pltpu.VMEM((R * LAT_C,), jnp.float32),
            pltpu.VMEM((R * HALF,), jnp.float32),
            pltpu.VMEM((R * HALF,), jnp.float32),
        ],
        compiler_params=pltpu.CompilerParams(needs_layout_passes=False),
    )
    def sc_split(x_hbm, even_hbm, odd_hbm, in_v, ev_v, od_v):
        wid = lax.axis_index("s") * NC + lax.axis_index("c")
        base_row = wid * rows_per_w
        two_iota = lax.iota(jnp.int32, L) * 2

        def chunk_body(c, carry):
            row0 = base_row + c * R
            pltpu.sync_copy(x_hbm.at[pl.ds(row0 * LAT_C, R * LAT_C)], in_v)

            def row_body(i, carry2):
                p = lax.rem(i, 2)
                row_off = i * LAT_C

                for kk in range(n_vregs):
                    ce = two_iota + (2 * L * kk + p)
                    co = jnp.bitwise_and(ce + 1, LAT_C - 1)
                    ve = plsc.load_gather(in_v, [row_off + ce])
                    vo = plsc.load_gather(in_v, [row_off + co])
                    ev_v[pl.ds(i * HALF + kk * L, L)] = ve
                    od_v[pl.ds(i * HALF + kk * L, L)] = vo
                return carry2

            lax.fori_loop(0, R, row_body, 0)
            pltpu.sync_copy(ev_v, even_hbm.at[pl.ds(row0 * HALF, R * HALF)])
            pltpu.sync_copy(od_v, odd_hbm.at[pl.ds(row0 * HALF, R * HALF)])
            return carry

        lax.fori_loop(0, n_chunks, chunk_body, 0)

    return sc_split


def kernel(x):
    num_samples = x.shape[0]
    num_rows = num_samples * LAT_R
    x_flat = x.reshape(num_rows * LAT_C)
    even, odd = _make_sc_split(num_rows)(x_flat)
    shape = (num_samples, LAT_R, HALF)
    return (even.reshape(shape), odd.reshape(shape))


# speedup vs baseline: 1.9596x; 1.9596x over previous
import functools

import jax
import jax.numpy as jnp
from jax import lax
from jax.experimental import pallas as pl
from jax.experimental.pallas import tpu as pltpu
from jax.experimental.pallas import tpu_sc as plsc

LAT_R = 256
LAT_C = 256
HALF = LAT_C // 2


@functools.lru_cache(maxsize=None)
def _make_sc_split(num_rows: int):
    info = plsc.get_sparse_core_info()
    NC, NS, L = info.num_cores, info.num_subcores, info.num_lanes
    NW = NC * NS
    rows_per_w = num_rows // NW
    R = 64
    n_chunks = rows_per_w // R
    n_vregs = HALF // L

    mesh = plsc.VectorSubcoreMesh(core_axis_name="c", subcore_axis_name="s")

    @functools.partial(
        pl.kernel,
        mesh=mesh,
        out_type=(
            jax.ShapeDtypeStruct((num_rows * HALF,), jnp.float32),
            jax.ShapeDtypeStruct((num_rows * HALF,), jnp.float32),
        ),
        scratch_types=[
            pltpu.VMEM((R * LAT_C,), jnp.float32),
            pltpu.VMEM((R * LAT_C,), jnp.float32),
            pltpu.VMEM((R * HALF,), jnp.float32),
            pltpu.VMEM((R * HALF,), jnp.float32),
            pltpu.VMEM((R * HALF,), jnp.float32),
            pltpu.VMEM((R * HALF,), jnp.float32),
            pltpu.SemaphoreType.DMA,
            pltpu.SemaphoreType.DMA,
            pltpu.SemaphoreType.DMA,
            pltpu.SemaphoreType.DMA,
            pltpu.SemaphoreType.DMA,
            pltpu.SemaphoreType.DMA,
        ],
        compiler_params=pltpu.CompilerParams(needs_layout_passes=False),
    )
    def sc_split(
        x_hbm, even_hbm, odd_hbm,
        in_v0, in_v1, ev_v0, ev_v1, od_v0, od_v1,
        in_s0, in_s1, ev_s0, ev_s1, od_s0, od_s1,
    ):
        wid = lax.axis_index("s") * NC + lax.axis_index("c")
        base_row = wid * rows_per_w

        two_iota = lax.iota(jnp.int32, L) * 2
        ce_c = [two_iota + 2 * L * kk for kk in range(n_vregs)]
        co_c = [v + 1 for v in ce_c]
        cr_c = [jnp.bitwise_and(v + 2, LAT_C - 1) for v in ce_c]

        def in_slice(c):
            return x_hbm.at[pl.ds((base_row + c * R) * LAT_C, R * LAT_C)]

        def out_slice(hbm, c):
            return hbm.at[pl.ds((base_row + c * R) * HALF, R * HALF)]

        def compute_chunk(in_ref, ev_ref, od_ref):
            @plsc.parallel_loop(0, R // 2, 1, unroll=2)
            def _(rp):
                off_e = rp * (2 * LAT_C)
                off_o = off_e + LAT_C
                out_e = rp * (2 * HALF)
                out_o = out_e + HALF
                for kk in range(n_vregs):
                    ve = plsc.load_gather(in_ref, [ce_c[kk] + off_e])
                    vo = plsc.load_gather(in_ref, [co_c[kk] + off_e])
                    ev_ref[pl.ds(out_e + kk * L, L)] = ve
                    od_ref[pl.ds(out_e + kk * L, L)] = vo
                    ve2 = plsc.load_gather(in_ref, [co_c[kk] + off_o])
                    vo2 = plsc.load_gather(in_ref, [cr_c[kk] + off_o])
                    ev_ref[pl.ds(out_o + kk * L, L)] = ve2
                    od_ref[pl.ds(out_o + kk * L, L)] = vo2

        def step(c, in_ref, ev_ref, od_ref, in_sem, ev_sem, od_sem):
            pltpu.make_async_copy(in_slice(c), in_ref, in_sem).wait()

            @pl.when(c >= 2)
            def _():
                pltpu.make_async_copy(ev_ref, out_slice(even_hbm, c), ev_sem).wait()
                pltpu.make_async_copy(od_ref, out_slice(odd_hbm, c), od_sem).wait()

            compute_chunk(in_ref, ev_ref, od_ref)
            pltpu.async_copy(ev_ref, out_slice(even_hbm, c), ev_sem)
            pltpu.async_copy(od_ref, out_slice(odd_hbm, c), od_sem)

            @pl.when(c + 2 < n_chunks)
            def _():
                pltpu.async_copy(in_slice(c + 2), in_ref, in_sem)

        pltpu.async_copy(in_slice(0), in_v0, in_s0)
        pltpu.async_copy(in_slice(1), in_v1, in_s1)

        def g_body(g, carry):
            step(g * 2, in_v0, ev_v0, od_v0, in_s0, ev_s0, od_s0)
            step(g * 2 + 1, in_v1, ev_v1, od_v1, in_s1, ev_s1, od_s1)
            return carry

        lax.fori_loop(0, n_chunks // 2, g_body, 0)

        pltpu.make_async_copy(ev_v0, out_slice(even_hbm, 0), ev_s0).wait()
        pltpu.make_async_copy(od_v0, out_slice(odd_hbm, 0), od_s0).wait()
        pltpu.make_async_copy(ev_v1, out_slice(even_hbm, 1), ev_s1).wait()
        pltpu.make_async_copy(od_v1, out_slice(odd_hbm, 1), od_s1).wait()

    return sc_split


def kernel(x):
    num_samples = x.shape[0]
    num_rows = num_samples * LAT_R
    x_flat = x.reshape(num_rows * LAT_C)
    even, odd = _make_sc_split(num_rows)(x_flat)
    shape = (num_samples, LAT_R, HALF)
    return (even.reshape(shape), odd.reshape(shape))


# speedup vs baseline: 3.5653x; 1.8194x over previous
import functools

import jax
import jax.numpy as jnp
from jax import lax
from jax.experimental import pallas as pl
from jax.experimental.pallas import tpu as pltpu
from jax.experimental.pallas import tpu_sc as plsc

LAT_R = 256
LAT_C = 256
HALF = LAT_C // 2
TILE_R = 8
TILE_C = 128


@functools.lru_cache(maxsize=None)
def _make_sc_split(num_rows: int):
    info = plsc.get_sparse_core_info()
    NC, NS, L = info.num_cores, info.num_subcores, info.num_lanes
    NW = NC * NS
    rows_per_w = num_rows // NW
    R = 64
    TB = R // TILE_R * 2
    n_chunks = rows_per_w // R
    n_vregs = HALF // L

    mesh = plsc.VectorSubcoreMesh(core_axis_name="c", subcore_axis_name="s")

    @functools.partial(
        pl.kernel,
        mesh=mesh,
        out_type=(
            jax.ShapeDtypeStruct((num_rows * HALF,), jnp.float32),
            jax.ShapeDtypeStruct((num_rows * HALF,), jnp.float32),
        ),
        scratch_types=[
            pltpu.VMEM((TB, TILE_R, TILE_C), jnp.float32),
            pltpu.VMEM((TB, TILE_R, TILE_C), jnp.float32),
            pltpu.VMEM((R * HALF,), jnp.float32),
            pltpu.VMEM((R * HALF,), jnp.float32),
            pltpu.VMEM((R * HALF,), jnp.float32),
            pltpu.VMEM((R * HALF,), jnp.float32),
            pltpu.SemaphoreType.DMA,
            pltpu.SemaphoreType.DMA,
            pltpu.SemaphoreType.DMA,
            pltpu.SemaphoreType.DMA,
            pltpu.SemaphoreType.DMA,
            pltpu.SemaphoreType.DMA,
        ],
        compiler_params=pltpu.CompilerParams(needs_layout_passes=False),
    )
    def sc_split(
        x_hbm, even_hbm, odd_hbm,
        in_v0, in_v1, ev_v0, ev_v1, od_v0, od_v1,
        in_s0, in_s1, ev_s0, ev_s1, od_s0, od_s1,
    ):
        wid = lax.axis_index("s") * NC + lax.axis_index("c")
        base_row = wid * rows_per_w

        zero_v = jnp.zeros((L,), jnp.int32)
        two_iota = lax.iota(jnp.int32, L) * 2

        def colmap(c):
            return c + (TILE_R - 1) * TILE_C * (c >> 7)

        ce_c = [colmap(two_iota + 2 * L * kk) for kk in range(n_vregs)]
        co_c = [colmap(two_iota + 2 * L * kk + 1) for kk in range(n_vregs)]
        cr_c = [
            colmap(jnp.bitwise_and(two_iota + 2 * L * kk + 2, LAT_C - 1))
            for kk in range(n_vregs)
        ]

        def in_slice(c):
            return x_hbm.at[pl.ds((base_row + c * R) // TILE_R * 2, TB)]

        def out_slice(hbm, c):
            return hbm.at[pl.ds((base_row + c * R) * HALF, R * HALF)]

        def compute_chunk(in_ref, ev_ref, od_ref):
            @plsc.parallel_loop(0, R // 2, 1, unroll=2)
            def _(rp):
                base_e = (rp >> 2) * 2048 + (rp & 3) * 256
                base_o = base_e + TILE_C
                out_e = rp * (2 * HALF)
                out_o = out_e + HALF
                for kk in range(n_vregs):
                    ve = plsc.load_gather(in_ref, [zero_v, zero_v, ce_c[kk] + base_e])
                    vo = plsc.load_gather(in_ref, [zero_v, zero_v, co_c[kk] + base_e])
                    ev_ref[pl.ds(out_e + kk * L, L)] = ve
                    od_ref[pl.ds(out_e + kk * L, L)] = vo
                    ve2 = plsc.load_gather(in_ref, [zero_v, zero_v, co_c[kk] + base_o])
                    vo2 = plsc.load_gather(in_ref, [zero_v, zero_v, cr_c[kk] + base_o])
                    ev_ref[pl.ds(out_o + kk * L, L)] = ve2
                    od_ref[pl.ds(out_o + kk * L, L)] = vo2

        def step(c, in_ref, ev_ref, od_ref, in_sem, ev_sem, od_sem):
            pltpu.make_async_copy(in_slice(c), in_ref, in_sem).wait()

            @pl.when(c >= 2)
            def _():
                pltpu.make_async_copy(ev_ref, out_slice(even_hbm, c), ev_sem).wait()
                pltpu.make_async_copy(od_ref, out_slice(odd_hbm, c), od_sem).wait()

            compute_chunk(in_ref, ev_ref, od_ref)
            pltpu.async_copy(ev_ref, out_slice(even_hbm, c), ev_sem)
            pltpu.async_copy(od_ref, out_slice(odd_hbm, c), od_sem)

            @pl.when(c + 2 < n_chunks)
            def _():
                pltpu.async_copy(in_slice(c + 2), in_ref, in_sem)

        pltpu.async_copy(in_slice(0), in_v0, in_s0)
        pltpu.async_copy(in_slice(1), in_v1, in_s1)

        def g_body(g, carry):
            step(g * 2, in_v0, ev_v0, od_v0, in_s0, ev_s0, od_s0)
            step(g * 2 + 1, in_v1, ev_v1, od_v1, in_s1, ev_s1, od_s1)
            return carry

        lax.fori_loop(0, n_chunks // 2, g_body, 0)

        pltpu.make_async_copy(ev_v0, out_slice(even_hbm, 0), ev_s0).wait()
        pltpu.make_async_copy(od_v0, out_slice(odd_hbm, 0), od_s0).wait()
        pltpu.make_async_copy(ev_v1, out_slice(even_hbm, 1), ev_s1).wait()
        pltpu.make_async_copy(od_v1, out_slice(odd_hbm, 1), od_s1).wait()

    return sc_split


def kernel(x):
    num_samples = x.shape[0]
    num_rows = num_samples * LAT_R
    x_view = (
        x.reshape(num_rows // TILE_R, TILE_R, 2, TILE_C)
        .transpose(0, 2, 1, 3)
        .reshape(num_rows * 2 // TILE_R, TILE_R, TILE_C)
    )
    even, odd = _make_sc_split(num_rows)(x_view)
    shape = (num_samples, LAT_R, HALF)
    return (even.reshape(shape), odd.reshape(shape))


# speedup vs baseline: 3.6717x; 1.0298x over previous
import functools

import jax
import jax.numpy as jnp
from jax import lax
from jax.experimental import pallas as pl
from jax.experimental.pallas import tpu as pltpu
from jax.experimental.pallas import tpu_sc as plsc

LAT_R = 256
LAT_C = 256
HALF = LAT_C // 2
TILE_R = 8
TILE_C = 128


@functools.lru_cache(maxsize=None)
def _make_sc_split(num_rows: int):
    info = plsc.get_sparse_core_info()
    NC, NS, L = info.num_cores, info.num_subcores, info.num_lanes
    NW = NC * NS
    rows_per_w = num_rows // NW
    R = 64
    TB = R // TILE_R * 2
    n_chunks = rows_per_w // R
    n_vregs = HALF // L

    mesh = plsc.VectorSubcoreMesh(core_axis_name="c", subcore_axis_name="s")

    @functools.partial(
        pl.kernel,
        mesh=mesh,
        out_type=(
            jax.ShapeDtypeStruct((num_rows * HALF,), jnp.float32),
            jax.ShapeDtypeStruct((num_rows * HALF,), jnp.float32),
        ),
        scratch_types=[
            pltpu.VMEM((TB, TILE_R, TILE_C), jnp.float32),
            pltpu.VMEM((TB, TILE_R, TILE_C), jnp.float32),
            pltpu.VMEM((R * HALF,), jnp.float32),
            pltpu.VMEM((R * HALF,), jnp.float32),
            pltpu.VMEM((R * HALF,), jnp.float32),
            pltpu.VMEM((R * HALF,), jnp.float32),
            pltpu.SemaphoreType.DMA,
            pltpu.SemaphoreType.DMA,
            pltpu.SemaphoreType.DMA,
            pltpu.SemaphoreType.DMA,
            pltpu.SemaphoreType.DMA,
            pltpu.SemaphoreType.DMA,
        ],
        compiler_params=pltpu.CompilerParams(needs_layout_passes=False),
    )
    def sc_split(
        x_hbm, even_hbm, odd_hbm,
        in_v0, in_v1, ev_v0, ev_v1, od_v0, od_v1,
        in_s0, in_s1, ev_s0, ev_s1, od_s0, od_s1,
    ):
        wid = lax.axis_index("s") * NC + lax.axis_index("c")
        base_row = wid * rows_per_w

        zero_v = jnp.zeros((L,), jnp.int32)
        two_iota = lax.iota(jnp.int32, L) * 2

        def colmap(c):
            return c + (TILE_R - 1) * TILE_C * (c >> 7)

        ce_c = [colmap(two_iota + 2 * L * kk) for kk in range(n_vregs)]
        cr3_c = colmap(two_iota + 2 * L * 3 + 2)
        cr7_c = colmap(jnp.bitwise_and(two_iota + 2 * L * 7 + 2, LAT_C - 1))

        def in_slice(c):
            return x_hbm.at[pl.ds((base_row + c * R) // TILE_R * 2, TB)]

        def out_slice(hbm, c):
            return hbm.at[pl.ds((base_row + c * R) * HALF, R * HALF)]

        def compute_chunk(in_ref, ev_ref, od_ref):
            @plsc.parallel_loop(0, R // 2, 1, unroll=1)
            def _(rp):
                base_e = (rp >> 2) * 2048 + (rp & 3) * 256
                base_o = base_e + TILE_C
                out_e = rp * (2 * HALF)
                out_o = out_e + HALF
                for kk in range(n_vregs):
                    idx_e = ce_c[kk] + base_e
                    if kk == 3:
                        idx_r = cr3_c + base_o
                    elif kk == 7:
                        idx_r = cr7_c + base_o
                    else:
                        idx_r = idx_e + (TILE_C + 2)
                    ve = plsc.load_gather(in_ref, [zero_v, zero_v, idx_e])
                    vo = plsc.load_gather(in_ref, [zero_v, zero_v, idx_e + 1])
                    ev_ref[pl.ds(out_e + kk * L, L)] = ve
                    od_ref[pl.ds(out_e + kk * L, L)] = vo
                    ve2 = plsc.load_gather(in_ref, [zero_v, zero_v, idx_e + (TILE_C + 1)])
                    vo2 = plsc.load_gather(in_ref, [zero_v, zero_v, idx_r])
                    ev_ref[pl.ds(out_o + kk * L, L)] = ve2
                    od_ref[pl.ds(out_o + kk * L, L)] = vo2

        def step(c, in_ref, ev_ref, od_ref, in_sem, ev_sem, od_sem):
            pltpu.make_async_copy(in_slice(c), in_ref, in_sem).wait()

            @pl.when(c >= 2)
            def _():
                pltpu.make_async_copy(ev_ref, out_slice(even_hbm, c), ev_sem).wait()
                pltpu.make_async_copy(od_ref, out_slice(odd_hbm, c), od_sem).wait()

            compute_chunk(in_ref, ev_ref, od_ref)
            pltpu.async_copy(ev_ref, out_slice(even_hbm, c), ev_sem)
            pltpu.async_copy(od_ref, out_slice(odd_hbm, c), od_sem)

            @pl.when(c + 2 < n_chunks)
            def _():
                pltpu.async_copy(in_slice(c + 2), in_ref, in_sem)

        pltpu.async_copy(in_slice(0), in_v0, in_s0)
        pltpu.async_copy(in_slice(1), in_v1, in_s1)

        def g_body(g, carry):
            step(g * 2, in_v0, ev_v0, od_v0, in_s0, ev_s0, od_s0)
            step(g * 2 + 1, in_v1, ev_v1, od_v1, in_s1, ev_s1, od_s1)
            return carry

        lax.fori_loop(0, n_chunks // 2, g_body, 0)

        pltpu.make_async_copy(ev_v0, out_slice(even_hbm, 0), ev_s0).wait()
        pltpu.make_async_copy(od_v0, out_slice(odd_hbm, 0), od_s0).wait()
        pltpu.make_async_copy(ev_v1, out_slice(even_hbm, 1), ev_s1).wait()
        pltpu.make_async_copy(od_v1, out_slice(odd_hbm, 1), od_s1).wait()

    return sc_split


def kernel(x):
    num_samples = x.shape[0]
    num_rows = num_samples * LAT_R
    x_view = (
        x.reshape(num_rows // TILE_R, TILE_R, 2, TILE_C)
        .transpose(0, 2, 1, 3)
        .reshape(num_rows * 2 // TILE_R, TILE_R, TILE_C)
    )
    even, odd = _make_sc_split(num_rows)(x_view)
    shape = (num_samples, LAT_R, HALF)
    return (even.reshape(shape), odd.reshape(shape))


# speedup vs baseline: 3.6860x; 1.0039x over previous
import functools

import jax
import jax.numpy as jnp
from jax import lax
from jax.experimental import pallas as pl
from jax.experimental.pallas import tpu as pltpu
from jax.experimental.pallas import tpu_sc as plsc

LAT_R = 256
LAT_C = 256
HALF = LAT_C // 2
TILE_R = 8
TILE_C = 128


@functools.lru_cache(maxsize=None)
def _make_sc_split(num_rows: int):
    info = plsc.get_sparse_core_info()
    NC, NS, L = info.num_cores, info.num_subcores, info.num_lanes
    NW = NC * NS
    rows_per_w = num_rows // NW
    R = 64
    TB = R // TILE_R * 2
    n_chunks = rows_per_w // R
    n_vregs = HALF // L

    mesh = plsc.VectorSubcoreMesh(core_axis_name="c", subcore_axis_name="s")

    @functools.partial(
        pl.kernel,
        mesh=mesh,
        out_type=(
            jax.ShapeDtypeStruct((num_rows * HALF,), jnp.float32),
            jax.ShapeDtypeStruct((num_rows * HALF,), jnp.float32),
        ),
        scratch_types=[
            pltpu.VMEM((R * LAT_C,), jnp.float32),
            pltpu.VMEM((R * LAT_C,), jnp.float32),
            pltpu.VMEM((R * HALF,), jnp.float32),
            pltpu.VMEM((R * HALF,), jnp.float32),
            pltpu.VMEM((R * HALF,), jnp.float32),
            pltpu.VMEM((R * HALF,), jnp.float32),
            pltpu.SemaphoreType.DMA,
            pltpu.SemaphoreType.DMA,
            pltpu.SemaphoreType.DMA,
            pltpu.SemaphoreType.DMA,
            pltpu.SemaphoreType.DMA,
            pltpu.SemaphoreType.DMA,
        ],
        compiler_params=pltpu.CompilerParams(needs_layout_passes=False),
    )
    def sc_split(
        x_hbm, even_hbm, odd_hbm,
        in_v0, in_v1, ev_v0, ev_v1, od_v0, od_v1,
        in_s0, in_s1, ev_s0, ev_s1, od_s0, od_s1,
    ):
        wid = lax.axis_index("s") * NC + lax.axis_index("c")
        base_row = wid * rows_per_w

        two_iota = lax.iota(jnp.int32, L) * 2

        def colmap(c):
            return c + (TILE_R - 1) * TILE_C * (c >> 7)

        ce_c = [colmap(two_iota + 2 * L * kk) for kk in range(n_vregs)]
        cr3_c = colmap(two_iota + 2 * L * 3 + 2)
        cr7_c = colmap(jnp.bitwise_and(two_iota + 2 * L * 7 + 2, LAT_C - 1))

        def in_slice(c):
            return x_hbm.at[pl.ds((base_row + c * R) * LAT_C, R * LAT_C)]

        def out_slice(hbm, c):
            return hbm.at[pl.ds((base_row + c * R) * HALF, R * HALF)]

        def compute_chunk(in_ref, ev_ref, od_ref):
            @plsc.parallel_loop(0, R // 2, 1, unroll=1)
            def _(rp):
                base_e = (rp >> 2) * 2048 + (rp & 3) * 256
                base_o = base_e + TILE_C
                out_e = rp * (2 * HALF)
                out_o = out_e + HALF
                for kk in range(n_vregs):
                    idx_e = ce_c[kk] + base_e
                    if kk == 3:
                        idx_r = cr3_c + base_o
                    elif kk == 7:
                        idx_r = cr7_c + base_o
                    else:
                        idx_r = idx_e + (TILE_C + 2)
                    ve = plsc.load_gather(in_ref, [idx_e])
                    vo = plsc.load_gather(in_ref, [idx_e + 1])
                    ev_ref[pl.ds(out_e + kk * L, L)] = ve
                    od_ref[pl.ds(out_e + kk * L, L)] = vo
                    ve2 = plsc.load_gather(in_ref, [idx_e + (TILE_C + 1)])
                    vo2 = plsc.load_gather(in_ref, [idx_r])
                    ev_ref[pl.ds(out_o + kk * L, L)] = ve2
                    od_ref[pl.ds(out_o + kk * L, L)] = vo2

        def step(c, in_ref, ev_ref, od_ref, in_sem, ev_sem, od_sem):
            pltpu.make_async_copy(in_slice(c), in_ref, in_sem).wait()

            @pl.when(c >= 2)
            def _():
                pltpu.make_async_copy(ev_ref, out_slice(even_hbm, c), ev_sem).wait()
                pltpu.make_async_copy(od_ref, out_slice(odd_hbm, c), od_sem).wait()

            compute_chunk(in_ref, ev_ref, od_ref)
            pltpu.async_copy(ev_ref, out_slice(even_hbm, c), ev_sem)
            pltpu.async_copy(od_ref, out_slice(odd_hbm, c), od_sem)

            @pl.when(c + 2 < n_chunks)
            def _():
                pltpu.async_copy(in_slice(c + 2), in_ref, in_sem)

        pltpu.async_copy(in_slice(0), in_v0, in_s0)
        pltpu.async_copy(in_slice(1), in_v1, in_s1)

        def g_body(g, carry):
            step(g * 2, in_v0, ev_v0, od_v0, in_s0, ev_s0, od_s0)
            step(g * 2 + 1, in_v1, ev_v1, od_v1, in_s1, ev_s1, od_s1)
            return carry

        lax.fori_loop(0, n_chunks // 2, g_body, 0)

        pltpu.make_async_copy(ev_v0, out_slice(even_hbm, 0), ev_s0).wait()
        pltpu.make_async_copy(od_v0, out_slice(odd_hbm, 0), od_s0).wait()
        pltpu.make_async_copy(ev_v1, out_slice(even_hbm, 1), ev_s1).wait()
        pltpu.make_async_copy(od_v1, out_slice(odd_hbm, 1), od_s1).wait()

    return sc_split


def kernel(x):
    num_samples = x.shape[0]
    num_rows = num_samples * LAT_R
    x_view = (
        x.reshape(num_rows // TILE_R, TILE_R, 2, TILE_C)
        .transpose(0, 2, 1, 3)
        .reshape(num_rows * LAT_C)
    )
    even, odd = _make_sc_split(num_rows)(x_view)
    shape = (num_samples, LAT_R, HALF)
    return (even.reshape(shape), odd.reshape(shape))


# speedup vs baseline: 3.7079x; 1.0059x over previous
import functools

import jax
import jax.numpy as jnp
from jax import lax
from jax.experimental import pallas as pl
from jax.experimental.pallas import tpu as pltpu
from jax.experimental.pallas import tpu_sc as plsc

LAT_R = 256
LAT_C = 256
HALF = LAT_C // 2
TILE_R = 8
TILE_C = 128


@functools.lru_cache(maxsize=None)
def _make_sc_split(num_rows: int):
    info = plsc.get_sparse_core_info()
    NC, NS, L = info.num_cores, info.num_subcores, info.num_lanes
    NW = NC * NS
    rows_per_w = num_rows // NW
    R = 32
    NBUF = 4
    n_chunks = rows_per_w // R
    n_vregs = HALF // L

    mesh = plsc.VectorSubcoreMesh(core_axis_name="c", subcore_axis_name="s")

    @functools.partial(
        pl.kernel,
        mesh=mesh,
        out_type=(
            jax.ShapeDtypeStruct((num_rows * HALF,), jnp.float32),
            jax.ShapeDtypeStruct((num_rows * HALF,), jnp.float32),
        ),
        scratch_types=(
            [pltpu.VMEM((R * LAT_C,), jnp.float32)] * NBUF
            + [pltpu.VMEM((R * HALF,), jnp.float32)] * (2 * NBUF)
            + [pltpu.SemaphoreType.DMA] * (3 * NBUF)
        ),
        compiler_params=pltpu.CompilerParams(needs_layout_passes=False),
    )
    def sc_split(x_hbm, even_hbm, odd_hbm, *bufs):
        in_v = bufs[:NBUF]
        ev_v = bufs[NBUF : 2 * NBUF]
        od_v = bufs[2 * NBUF : 3 * NBUF]
        in_s = bufs[3 * NBUF : 4 * NBUF]
        ev_s = bufs[4 * NBUF : 5 * NBUF]
        od_s = bufs[5 * NBUF : 6 * NBUF]
        wid = lax.axis_index("s") * NC + lax.axis_index("c")
        base_row = wid * rows_per_w

        two_iota = lax.iota(jnp.int32, L) * 2

        def colmap(c):
            return c + (TILE_R - 1) * TILE_C * (c >> 7)

        ce_c = [colmap(two_iota + 2 * L * kk) for kk in range(n_vregs)]
        cr3_c = colmap(two_iota + 2 * L * 3 + 2)
        cr7_c = colmap(jnp.bitwise_and(two_iota + 2 * L * 7 + 2, LAT_C - 1))

        def in_slice(c):
            return x_hbm.at[pl.ds((base_row + c * R) * LAT_C, R * LAT_C)]

        def out_slice(hbm, c):
            return hbm.at[pl.ds((base_row + c * R) * HALF, R * HALF)]

        def compute_chunk(in_ref, ev_ref, od_ref):
            @plsc.parallel_loop(0, R // 2, 1, unroll=1)
            def _(rp):
                base_e = (rp >> 2) * 2048 + (rp & 3) * 256
                base_o = base_e + TILE_C
                out_e = rp * (2 * HALF)
                out_o = out_e + HALF
                for kk in range(n_vregs):
                    idx_e = ce_c[kk] + base_e
                    if kk == 3:
                        idx_r = cr3_c + base_o
                    elif kk == 7:
                        idx_r = cr7_c + base_o
                    else:
                        idx_r = idx_e + (TILE_C + 2)
                    ve = plsc.load_gather(in_ref, [idx_e])
                    vo = plsc.load_gather(in_ref, [idx_e + 1])
                    ev_ref[pl.ds(out_e + kk * L, L)] = ve
                    od_ref[pl.ds(out_e + kk * L, L)] = vo
                    ve2 = plsc.load_gather(in_ref, [idx_e + (TILE_C + 1)])
                    vo2 = plsc.load_gather(in_ref, [idx_r])
                    ev_ref[pl.ds(out_o + kk * L, L)] = ve2
                    od_ref[pl.ds(out_o + kk * L, L)] = vo2

        def step(c, b):
            pltpu.make_async_copy(in_slice(c), in_v[b], in_s[b]).wait()

            @pl.when(c >= NBUF)
            def _():
                pltpu.make_async_copy(ev_v[b], out_slice(even_hbm, c), ev_s[b]).wait()
                pltpu.make_async_copy(od_v[b], out_slice(odd_hbm, c), od_s[b]).wait()

            compute_chunk(in_v[b], ev_v[b], od_v[b])

            @pl.when(c + NBUF < n_chunks)
            def _():
                pltpu.async_copy(in_slice(c + NBUF), in_v[b], in_s[b])

            pltpu.async_copy(ev_v[b], out_slice(even_hbm, c), ev_s[b])
            pltpu.async_copy(od_v[b], out_slice(odd_hbm, c), od_s[b])

        for b in range(NBUF):
            pltpu.async_copy(in_slice(b), in_v[b], in_s[b])

        def g_body(g, carry):
            for b in range(NBUF):
                step(g * NBUF + b, b)
            return carry

        lax.fori_loop(0, n_chunks // NBUF, g_body, 0)

        for b in range(NBUF):
            pltpu.make_async_copy(ev_v[b], out_slice(even_hbm, 0), ev_s[b]).wait()
            pltpu.make_async_copy(od_v[b], out_slice(odd_hbm, 0), od_s[b]).wait()

    return sc_split


def kernel(x):
    num_samples = x.shape[0]
    num_rows = num_samples * LAT_R
    x_view = (
        x.reshape(num_rows // TILE_R, TILE_R, 2, TILE_C)
        .transpose(0, 2, 1, 3)
        .reshape(num_rows * LAT_C)
    )
    even, odd = _make_sc_split(num_rows)(x_view)
    shape = (num_samples, LAT_R, HALF)
    return (even.reshape(shape), odd.reshape(shape))
